# baseline (speedup 1.0000x reference)
import functools

import jax
import jax.numpy as jnp
from jax import lax
from jax.experimental import pallas as pl
from jax.experimental.pallas import tpu as pltpu
from jax.experimental.pallas import tpu_sc as plsc

NC, NS = 2, 16
NW = NC * NS
CHUNK = 512
HALF = CHUNK // 2
IDX_PER_STREAM = 128
NBUF = 2
L = 16


def kernel(x, W):
    B = x.size
    D = W.shape[1]
    xf = x.reshape(B)

    V = W.shape[0]
    T = jnp.concatenate(
        [
            jnp.broadcast_to(W[:, None, :], (V, V, D)),
            jnp.broadcast_to(W[None, :, :], (V, V, D)),
        ],
        axis=-1,
    ).reshape(V * V, 2 * D)

    b_per_w = B // NW
    n_iter = b_per_w // CHUNK
    n_streams = HALF // IDX_PER_STREAM

    mesh = plsc.VectorSubcoreMesh(core_axis_name="c", subcore_axis_name="s")

    @functools.partial(
        pl.kernel,
        out_type=jax.ShapeDtypeStruct((B, D), jnp.float32),
        mesh=mesh,
        scratch_types=[
            pltpu.VMEM_SHARED((V * V, 2 * D), jnp.float32),
            pltpu.VMEM((NBUF, CHUNK), jnp.int32),
            pltpu.VMEM((NBUF, HALF), jnp.int32),
            pltpu.VMEM((NBUF, HALF, 2 * D), jnp.float32),
            pltpu.SemaphoreType.DMA,
            pltpu.SemaphoreType.DMA,
            pltpu.SemaphoreType.DMA,
            pltpu.SemaphoreType.DMA,
            pltpu.SemaphoreType.DMA,
            pltpu.SemaphoreType.DMA,
        ],
        compiler_params=pltpu.CompilerParams(use_tc_tiling_on_sc=False),
    )
    def emb(x_hbm, t_hbm, out_hbm, t_sh, idx_v, pidx_v, rows_v, gsem0, gsem1,
            isem0, isem1, osem0, osem1):
        sid = lax.axis_index("s")
        wid = sid * NC + lax.axis_index("c")
        base = wid * b_per_w
        gsems = (gsem0, gsem1)
        isems = (isem0, isem1)
        osems = (osem0, osem1)

        @pl.when(sid == 0)
        def _():
            pltpu.sync_copy(t_hbm, t_sh)
        plsc.subcore_barrier()

        def idx_in(it, b):
            off = pl.multiple_of(base + it * CHUNK, CHUNK)
            return pltpu.make_async_copy(
                x_hbm.at[pl.ds(off, CHUNK)], idx_v.at[b], isems[b])

        def rows_out(it, b):
            off = pl.multiple_of(base + it * CHUNK, CHUNK)
            return [
                pltpu.make_async_copy(
                    rows_v.at[b].at[:, pl.ds(0, D)],
                    out_hbm.at[pl.ds(off, HALF)],
                    osems[b],
                ),
                pltpu.make_async_copy(
                    rows_v.at[b].at[:, pl.ds(D, D)],
                    out_hbm.at[pl.ds(off + HALF, HALF)],
                    osems[b],
                ),
            ]

        def gathers(b):
            return [
                pltpu.make_async_copy(
                    t_sh.at[pidx_v.at[b].at[pl.ds(j * IDX_PER_STREAM,
                                                  IDX_PER_STREAM)]],
                    rows_v.at[b].at[pl.ds(j * IDX_PER_STREAM, IDX_PER_STREAM)],
                    gsems[b],
                )
                for j in range(n_streams)
            ]

        for b in range(NBUF):
            idx_in(b, b).start()

        def half(it, b):
            @pl.when(it >= NBUF)
            def _():
                for c in rows_out(it - NBUF, b):
                    c.wait()
            idx_in(it, b).wait()
            for g in range(HALF // L):
                lo = idx_v[b, pl.ds(g * L, L)]
                hi = idx_v[b, pl.ds(HALF + g * L, L)]
                pidx_v[b, pl.ds(g * L, L)] = lo * V + hi
            for c in gathers(b):
                c.start()
            @pl.when(it >= 1)
            def _():
                for c in gathers(1 - b):
                    c.wait()
                for c in rows_out(it - 1, 1 - b):
                    c.start()
                @pl.when(it + 1 < n_iter)
                def _():
                    idx_in(it + 1, 1 - b).start()

        def body(i2, _):
            it = i2 * NBUF
            for b in range(NBUF):
                half(it + b, b)
            return ()

        lax.fori_loop(0, n_iter // NBUF, body, ())
        last_b = (n_iter - 1) % NBUF
        for c in gathers(last_b):
            c.wait()
        for c in rows_out(n_iter - 1, last_b):
            c.start()
        for c in rows_out(n_iter - 2, 1 - last_b):
            c.wait()
        for c in rows_out(n_iter - 1, last_b):
            c.wait()

    out = emb(xf, T)
    return out.reshape(*x.shape, D)


# speedup vs baseline: 1.1133x; 1.1133x over previous
import functools

import jax
import jax.numpy as jnp
from jax import lax
from jax.experimental import pallas as pl
from jax.experimental.pallas import tpu as pltpu
from jax.experimental.pallas import tpu_sc as plsc

NC, NS = 2, 16
NW = NC * NS
CHUNK = 512
IDX_PER_STREAM = 128
NBUF = 2


def kernel(x, W):
    B = x.size
    D = W.shape[1]
    xf = x.reshape(B)

    b_per_w = B // NW
    n_iter = b_per_w // CHUNK
    n_streams = CHUNK // IDX_PER_STREAM

    mesh = plsc.VectorSubcoreMesh(core_axis_name="c", subcore_axis_name="s")

    @functools.partial(
        pl.kernel,
        out_type=jax.ShapeDtypeStruct((B, D), jnp.float32),
        mesh=mesh,
        scratch_types=[
            pltpu.VMEM_SHARED((10, D), jnp.float32),
            pltpu.VMEM((NBUF, CHUNK), jnp.int32),
            pltpu.VMEM((NBUF, CHUNK, D), jnp.float32),
            pltpu.SemaphoreType.DMA,
            pltpu.SemaphoreType.DMA,
            pltpu.SemaphoreType.DMA,
            pltpu.SemaphoreType.DMA,
            pltpu.SemaphoreType.DMA,
            pltpu.SemaphoreType.DMA,
        ],
        compiler_params=pltpu.CompilerParams(use_tc_tiling_on_sc=False),
    )
    def emb(x_hbm, w_hbm, out_hbm, w_sh, idx_v, rows_v, gsem0, gsem1,
            isem0, isem1, osem0, osem1):
        sid = lax.axis_index("s")
        wid = sid * NC + lax.axis_index("c")
        base = wid * b_per_w
        gsems = (gsem0, gsem1)
        isems = (isem0, isem1)
        osems = (osem0, osem1)

        @pl.when(sid == 0)
        def _():
            pltpu.sync_copy(w_hbm, w_sh)
        plsc.subcore_barrier()

        def idx_in(it, b):
            off = pl.multiple_of(base + it * CHUNK, CHUNK)
            return pltpu.make_async_copy(
                x_hbm.at[pl.ds(off, CHUNK)], idx_v.at[b], isems[b])

        def rows_out(it, b):
            off = pl.multiple_of(base + it * CHUNK, CHUNK)
            return pltpu.make_async_copy(
                rows_v.at[b], out_hbm.at[pl.ds(off, CHUNK)], osems[b])

        def gathers(b):
            return [
                pltpu.make_async_copy(
                    w_sh.at[idx_v.at[b].at[pl.ds(j * IDX_PER_STREAM,
                                                 IDX_PER_STREAM)]],
                    rows_v.at[b].at[pl.ds(j * IDX_PER_STREAM, IDX_PER_STREAM)],
                    gsems[b],
                )
                for j in range(n_streams)
            ]

        for b in range(NBUF):
            idx_in(b, b).start()

        def half(it, b):
            @pl.when(it >= NBUF)
            def _():
                rows_out(it - NBUF, b).wait()
            idx_in(it, b).wait()
            for c in gathers(b):
                c.start()
            @pl.when(it >= 1)
            def _():
                for c in gathers(1 - b):
                    c.wait()
                rows_out(it - 1, 1 - b).start()
                @pl.when(it + 1 < n_iter)
                def _():
                    idx_in(it + 1, 1 - b).start()

        def body(i2, _):
            it = i2 * NBUF
            for b in range(NBUF):
                half(it + b, b)
            return ()

        lax.fori_loop(0, n_iter // NBUF, body, ())
        last_b = (n_iter - 1) % NBUF
        for c in gathers(last_b):
            c.wait()
        rows_out(n_iter - 1, last_b).start()
        rows_out(n_iter - 2, 1 - last_b).wait()
        rows_out(n_iter - 1, last_b).wait()

    out = emb(xf, W)
    return out.reshape(*x.shape, D)


# speedup vs baseline: 1.1383x; 1.0225x over previous
import functools

import jax
import jax.numpy as jnp
from jax import lax
from jax.experimental import pallas as pl
from jax.experimental.pallas import tpu as pltpu
from jax.experimental.pallas import tpu_sc as plsc

NC, NS = 2, 16
NW = NC * NS
CHUNK = 512
PAIRS = CHUNK // 2
IDX_PER_STREAM = 128
NBUF = 2
L = 16


def kernel(x, W):
    B = x.size
    V, D = W.shape
    xf = x.reshape(B)

    T = jnp.concatenate(
        [
            jnp.broadcast_to(W[:, None, :], (V, V, D)),
            jnp.broadcast_to(W[None, :, :], (V, V, D)),
        ],
        axis=-1,
    ).reshape(V * V, 2 * D)

    b_per_w = B // NW
    n_iter = b_per_w // CHUNK
    n_streams = PAIRS // IDX_PER_STREAM

    mesh = plsc.VectorSubcoreMesh(core_axis_name="c", subcore_axis_name="s")

    @functools.partial(
        pl.kernel,
        out_type=jax.ShapeDtypeStruct((B // 2, 2 * D), jnp.float32),
        mesh=mesh,
        scratch_types=[
            pltpu.VMEM_SHARED((V * V, 2 * D), jnp.float32),
            pltpu.VMEM((NBUF, CHUNK), jnp.int32),
            pltpu.VMEM((NBUF, PAIRS), jnp.int32),
            pltpu.VMEM((NBUF, PAIRS, 2 * D), jnp.float32),
            pltpu.SemaphoreType.DMA,
            pltpu.SemaphoreType.DMA,
            pltpu.SemaphoreType.DMA,
            pltpu.SemaphoreType.DMA,
            pltpu.SemaphoreType.DMA,
            pltpu.SemaphoreType.DMA,
        ],
        compiler_params=pltpu.CompilerParams(use_tc_tiling_on_sc=True),
    )
    def emb(x_hbm, t_hbm, out_hbm, t_sh, idx_v, pidx_v, rows_v, gsem0, gsem1,
            isem0, isem1, osem0, osem1):
        sid = lax.axis_index("s")
        wid = sid * NC + lax.axis_index("c")
        base = wid * b_per_w
        gsems = (gsem0, gsem1)
        isems = (isem0, isem1)
        osems = (osem0, osem1)

        @pl.when(sid == 0)
        def _():
            pltpu.sync_copy(t_hbm, t_sh)
        plsc.subcore_barrier()

        lanes = lax.iota(jnp.int32, L)
        perm_e = (lanes * 2) % L
        perm_o = perm_e + 1
        low_half = lanes < (L // 2)

        gdn = lax.GatherDimensionNumbers(
            offset_dims=(), collapsed_slice_dims=(0,), start_index_map=(0,))

        def vperm(v, idx):
            return lax.gather(v, idx[:, None], gdn, slice_sizes=(1,),
                              mode=lax.GatherScatterMode.PROMISE_IN_BOUNDS)

        def idx_in(it, b):
            off = pl.multiple_of(base + it * CHUNK, CHUNK)
            return pltpu.make_async_copy(
                x_hbm.at[pl.ds(off, CHUNK)], idx_v.at[b], isems[b])

        def rows_out(it, b):
            off2 = pl.multiple_of((base + it * CHUNK) // 2, PAIRS)
            return pltpu.make_async_copy(
                rows_v.at[b], out_hbm.at[pl.ds(off2, PAIRS)], osems[b])

        def gathers(b):
            return [
                pltpu.make_async_copy(
                    t_sh.at[pidx_v.at[b].at[pl.ds(j * IDX_PER_STREAM,
                                                  IDX_PER_STREAM)]],
                    rows_v.at[b].at[pl.ds(j * IDX_PER_STREAM, IDX_PER_STREAM)],
                    gsems[b],
                )
                for j in range(n_streams)
            ]

        for b in range(NBUF):
            idx_in(b, b).start()

        def half(it, b):
            @pl.when(it >= NBUF)
            def _():
                rows_out(it - NBUF, b).wait()
            idx_in(it, b).wait()
            for g in range(PAIRS // L):
                v0 = idx_v[b, pl.ds(2 * g * L, L)]
                v1 = idx_v[b, pl.ds((2 * g + 1) * L, L)]
                ev = jnp.where(low_half, vperm(v0, perm_e), vperm(v1, perm_e))
                od = jnp.where(low_half, vperm(v0, perm_o), vperm(v1, perm_o))
                pidx_v[b, pl.ds(g * L, L)] = ev * V + od
            for c in gathers(b):
                c.start()
            @pl.when(it >= 1)
            def _():
                for c in gathers(1 - b):
                    c.wait()
                rows_out(it - 1, 1 - b).start()
                @pl.when(it + 1 < n_iter)
                def _():
                    idx_in(it + 1, 1 - b).start()

        def body(i2, _):
            it = i2 * NBUF
            for b in range(NBUF):
                half(it + b, b)
            return ()

        lax.fori_loop(0, n_iter // NBUF, body, ())
        last_b = (n_iter - 1) % NBUF
        for c in gathers(last_b):
            c.wait()
        rows_out(n_iter - 1, last_b).start()
        rows_out(n_iter - 2, 1 - last_b).wait()
        rows_out(n_iter - 1, last_b).wait()

    out = emb(xf, T)
    return out.reshape(*x.shape, D)
